# baseline (speedup 1.0000x reference)
import functools

import jax
import jax.numpy as jnp
from jax import lax
from jax.experimental import pallas as pl
from jax.experimental.pallas import tpu as pltpu
from jax.experimental.pallas import tpu_sc as plsc

BATCH = 64
HEIGHT = 32
WIDTH = 32
EMBED_DIM = 256
MAX_POS = 50

_MESH = plsc.VectorSubcoreMesh(core_axis_name="c", subcore_axis_name="s")


def _sc_body(row_hbm, col_hbm, out_hbm, chunk_v, sem):
    c = lax.axis_index("c")
    s = lax.axis_index("s")
    h = s * 2 + c
    pltpu.sync_copy(col_hbm.at[pl.ds(0, WIDTH), :], chunk_v.at[:, pl.ds(0, EMBED_DIM)])
    for w in range(WIDTH):
        pltpu.sync_copy(row_hbm.at[h], chunk_v.at[w, pl.ds(EMBED_DIM, EMBED_DIM)])
    copies = []
    for b in range(BATCH):
        copies.append(
            pltpu.async_copy(
                chunk_v, out_hbm.at[b, pl.ds(h * WIDTH, WIDTH), :], sem
            )
        )
    for cp in copies:
        cp.wait()


_sc_kernel = functools.partial(
    pl.kernel,
    mesh=_MESH,
    out_type=jax.ShapeDtypeStruct((BATCH, HEIGHT * WIDTH, 2 * EMBED_DIM), jnp.float32),
    scratch_types=[
        pltpu.VMEM((WIDTH, 2 * EMBED_DIM), jnp.float32),
        pltpu.SemaphoreType.DMA,
    ],
)(_sc_body)


def kernel(row_embeddings, column_embeddings):
    return _sc_kernel(row_embeddings, column_embeddings)
